# Initial kernel scaffold; baseline (speedup 1.0000x reference)
import jax
import jax.numpy as jnp
from jax.experimental import pallas as pl


def kernel(X, edge_index, edge_weight, edge_attr, Wp, bp, Ws1, bs1, Ws2, bs2, Ws3, bs3, Wt0, Wt1, Wt2, Wt3, Wt4, Wt5):
    raise NotImplementedError("write your pallas kernel here")


# speedup vs baseline: 26.7669x; 26.7669x over previous
import functools
import math

import numpy as np
import jax
import jax.numpy as jnp
from jax import lax
from jax.experimental import pallas as pl
from jax.experimental.pallas import tpu as pltpu
from jax.experimental.pallas import tpu_sc as plsc

N = 10000
E = 160000
C = 32
NQ = 9
CUTOFF = 5.0

NCORES = 2
NSUB = 16
HALF = N // NCORES
TPT = (HALF + NSUB - 1) // NSUB
PAD_ROWS = TPT * NSUB
DUMMY = PAD_ROWS
ACC_ROWS = PAD_ROWS + 1
EPT = E // NSUB
CH = 80
NCHUNK = EPT // CH

BN = 500
BE = 2000

GRP = (0, 1, 1, 1, 2, 2, 2, 2, 2)
PERM = np.array([3 * c + g for g in range(3) for c in range(C)], dtype=np.int32)


def _decomp(y):
    lam = (y[0] + y[4] + y[8]) * (1.0 / 3.0)
    a1 = (y[1] - y[3]) * 0.5
    a2 = (y[2] - y[6]) * 0.5
    a3 = (y[5] - y[7]) * 0.5
    s0 = y[0] - lam
    s1 = (y[1] + y[3]) * 0.5
    s2 = (y[2] + y[6]) * 0.5
    s3 = y[4] - lam
    s4 = (y[5] + y[7]) * 0.5
    return [lam, a1, a2, a3, s0, s1, s2, s3, s4]


def _recon(c):
    lam, a1, a2, a3, s0, s1, s2, s3, s4 = c
    return [lam + s0, s1 + a1, s2 + a2,
            s1 - a1, lam + s3, s4 + a3,
            s2 - a2, s4 - a3, lam - s0 - s3]


def _node_pre_body(x_ref, wp_ref, bp_ref, wt0_ref, wt1_ref, wt2_ref,
                   xn_ref, g_ref):
    wp = wp_ref[...]
    bp = bp_ref[...]
    y = []
    for q in range(NQ):
        xq = x_ref[:, q, :]
        y.append(jnp.dot(xq, wp.T, preferred_element_type=jnp.float32) + bp)
    tn = y[0] * y[0]
    for q in range(1, NQ):
        tn = tn + y[q] * y[q]
    inv = 1.0 / (jnp.maximum(tn, 0.01) + 1.0)
    y = [v * inv for v in y]
    for q in range(NQ):
        xn_ref[:, q, :] = y[q]
    comps = _decomp(y)
    wts = [wt0_ref[...], wt1_ref[...], wt2_ref[...]]
    for q in range(NQ):
        g_ref[:, q, :] = jnp.dot(comps[q], wts[GRP[q]].T,
                                 preferred_element_type=jnp.float32)


def _node_pre(xr, wp, bp, wt0, wt1, wt2):
    w_spec = pl.BlockSpec((C, C), lambda i: (0, 0))
    b_spec = pl.BlockSpec((1, C), lambda i: (0, 0))
    n_spec = pl.BlockSpec((BN, NQ, C), lambda i: (i, 0, 0))
    return pl.pallas_call(
        _node_pre_body,
        grid=(N // BN,),
        in_specs=[n_spec, w_spec, b_spec, w_spec, w_spec, w_spec],
        out_specs=[n_spec, n_spec],
        out_shape=[jax.ShapeDtypeStruct((N, NQ, C), jnp.float32),
                   jax.ShapeDtypeStruct((N, NQ, C), jnp.float32)],
    )(xr, wp, bp, wt0, wt1, wt2)


def _edge_mlp_body(ea_ref, ew_ref, w1_ref, b1_ref, w2_ref, b2_ref,
                   w3_ref, b3_ref, out_ref):
    h = ea_ref[...]
    h = jax.nn.silu(jnp.dot(h, w1_ref[...].T,
                            preferred_element_type=jnp.float32) + b1_ref[...])
    h = jax.nn.silu(jnp.dot(h, w2_ref[...].T,
                            preferred_element_type=jnp.float32) + b2_ref[...])
    h = jax.nn.silu(jnp.dot(h, w3_ref[...].T,
                            preferred_element_type=jnp.float32) + b3_ref[...])
    d = ew_ref[...]
    cc = 0.5 * (jnp.cos(d * (math.pi / CUTOFF)) + 1.0)
    cc = jnp.where(d < CUTOFF, cc, 0.0)
    out_ref[...] = h * cc


def _edge_mlp(ea, ew, w1, b1, w2, b2, w3, b3):
    def wspec(r, c):
        return pl.BlockSpec((r, c), lambda i: (0, 0))
    return pl.pallas_call(
        _edge_mlp_body,
        grid=(E // BE,),
        in_specs=[pl.BlockSpec((BE, NRBF_), lambda i: (i, 0)),
                  pl.BlockSpec((BE, 1), lambda i: (i, 0)),
                  wspec(C, NRBF_), wspec(1, C),
                  wspec(2 * C, C), wspec(1, 2 * C),
                  wspec(3 * C, 2 * C), wspec(1, 3 * C)],
        out_specs=pl.BlockSpec((BE, 3 * C), lambda i: (i, 0)),
        out_shape=jax.ShapeDtypeStruct((E, 3 * C), jnp.float32),
    )(ea, ew, w1, b1, w2, b2, w3, b3)


NRBF_ = 32


def _mp_body(g_hbm, ea_hbm, src_hbm, dst_hbm, z_hbm, out_hbm,
             src_v, dst_v, idx_v, ea_v, rows_v, acc_sh, sem):
    cid = lax.axis_index("c")
    sid = lax.axis_index("s")
    base_node = cid * HALF

    pltpu.sync_copy(z_hbm, acc_sh.at[pl.ds(sid * TPT, TPT)])

    @pl.when(sid == 0)
    def _zero_dummy():
        pltpu.sync_copy(z_hbm.at[pl.ds(0, 1)], acc_sh.at[pl.ds(DUMMY, 1)])

    plsc.subcore_barrier()

    def chunk_body(ch, _):
        ebase = sid * EPT + ch * CH
        pltpu.sync_copy(src_hbm.at[pl.ds(ebase, CH)], src_v)
        pltpu.sync_copy(dst_hbm.at[pl.ds(ebase, CH)], dst_v)
        pltpu.sync_copy(ea_hbm.at[pl.ds(ebase, CH)], ea_v)
        pltpu.async_copy(g_hbm.at[dst_v], rows_v, sem).wait()

        def idx_body(i, _):
            s16 = src_v[pl.ds(i * 16, 16)]
            rel = s16 - base_node
            owned = (rel >= 0) & (rel < HALF)
            idx_v[pl.ds(i * 16, 16)] = jnp.where(owned, rel, DUMMY)
            return 0

        lax.fori_loop(0, CH // 16, idx_body, 0)

        def edge_body(e, _):
            for q in range(NQ):
                for t in range(C // 16):
                    m = ea_v[e, pl.ds(GRP[q] * C + t * 16, 16)]
                    r = rows_v[e, q, pl.ds(t * 16, 16)]
                    rows_v[e, q, pl.ds(t * 16, 16)] = r * m
            return 0

        lax.fori_loop(0, CH, edge_body, 0)
        pltpu.sync_copy(rows_v, acc_sh.at[idx_v], add=True)
        return 0

    lax.fori_loop(0, NCHUNK, chunk_body, 0)
    plsc.subcore_barrier()
    r0 = sid * TPT
    pltpu.sync_copy(acc_sh.at[pl.ds(r0, TPT)],
                    out_hbm.at[cid, pl.ds(r0, TPT)])


def _mp_sc(g, ea, src, dst, zrows):
    mesh = plsc.VectorSubcoreMesh(core_axis_name="c", subcore_axis_name="s")
    fn = functools.partial(
        pl.kernel,
        mesh=mesh,
        compiler_params=pltpu.CompilerParams(use_tc_tiling_on_sc=False),
        out_type=jax.ShapeDtypeStruct((NCORES, PAD_ROWS, NQ, C), jnp.float32),
        scratch_types=[
            pltpu.VMEM((CH,), jnp.int32),
            pltpu.VMEM((CH,), jnp.int32),
            pltpu.VMEM((CH,), jnp.int32),
            pltpu.VMEM((CH, 3 * C), jnp.float32),
            pltpu.VMEM((CH, NQ, C), jnp.float32),
            pltpu.VMEM_SHARED((ACC_ROWS, NQ, C), jnp.float32),
            pltpu.SemaphoreType.DMA,
        ],
    )(_mp_body)
    return fn(g, ea, src, dst, zrows)


def _node_post_body(m_ref, g_ref, xn_ref, wt3_ref, wt4_ref, wt5_ref, out_ref):
    mc = [m_ref[0, :, q, :] for q in range(NQ)]
    gc = [g_ref[:, q, :] for q in range(NQ)]
    m = _recon(mc)
    yy = _recon(gc)
    f = []
    for i in range(3):
        for j in range(3):
            acc = None
            for k in range(3):
                term = m[i * 3 + k] * yy[k * 3 + j] + yy[i * 3 + k] * m[k * 3 + j]
                acc = term if acc is None else acc + term
            f.append(acc)
    tn = f[0] * f[0]
    for q in range(1, NQ):
        tn = tn + f[q] * f[q]
    inv = 1.0 / (jnp.maximum(tn, 0.01) + 1.0)
    comps = _decomp(f)
    comps = [v * inv for v in comps]
    wts = [wt3_ref[...], wt4_ref[...], wt5_ref[...]]
    mixed = [jnp.dot(comps[q], wts[GRP[q]].T,
                     preferred_element_type=jnp.float32) for q in range(NQ)]
    dx = _recon(mixed)
    for q in range(NQ):
        out_ref[:, q, :] = xn_ref[:, q, :] + dx[q]


def _node_post(msgp, g, xn, wt3, wt4, wt5):
    w_spec = pl.BlockSpec((C, C), lambda i: (0, 0))
    n_spec = pl.BlockSpec((BN, NQ, C), lambda i: (i, 0, 0))
    nb_half = HALF // BN
    m_spec = pl.BlockSpec((1, BN, NQ, C),
                          lambda i: (i // nb_half, i % nb_half, 0, 0))
    return pl.pallas_call(
        _node_post_body,
        grid=(N // BN,),
        in_specs=[m_spec, n_spec, n_spec, w_spec, w_spec, w_spec],
        out_specs=n_spec,
        out_shape=jax.ShapeDtypeStruct((N, NQ, C), jnp.float32),
    )(msgp, g, xn, wt3, wt4, wt5)


def kernel(X, edge_index, edge_weight, edge_attr, Wp, bp, Ws1, bs1, Ws2, bs2,
           Ws3, bs3, Wt0, Wt1, Wt2, Wt3, Wt4, Wt5):
    xr = X.reshape(N, C, NQ).transpose(0, 2, 1)
    xn, g = _node_pre(xr, Wp, bp.reshape(1, C), Wt0, Wt1, Wt2)
    perm = jnp.asarray(PERM)
    ea = _edge_mlp(edge_attr, edge_weight.reshape(E, 1),
                   Ws1, bs1.reshape(1, C), Ws2, bs2.reshape(1, 2 * C),
                   Ws3[perm], bs3[perm].reshape(1, 3 * C))
    src = edge_index[0]
    dst = edge_index[1]
    zrows = jnp.zeros((TPT, NQ, C), jnp.float32)
    msgp = _mp_sc(g, ea, src, dst, zrows)
    out = _node_post(msgp, g, xn, Wt3, Wt4, Wt5)
    return out.transpose(0, 2, 1).reshape(N, C, 3, 3)
